# baseline (speedup 1.0000x reference)
import jax
import jax.numpy as jnp
import numpy as np
from jax.experimental import pallas as pl

B, N, D_IN = 2, 96, 32
MUL = 16
NB, H = 3, 64
D_MID = 3 * MUL
D_OUT = 16
PB = N * N
BN = B * N
SCALE = 1.0 / np.sqrt(float(D_IN) * float(N))
C1 = float(np.exp(-1.0))
C3 = float(np.exp(-3.0))


def _sig(x):
    return 1.0 / (1.0 + jnp.exp(-x))


def _swish(x):
    return x * _sig(x)


def _net_kernel(g6, x0t, sel,
                w0_1, b0_1, w1_1, b1_1, w2t_1, bn1g, bn1b,
                w0_2, b0_2, w1_2, b1_2, w2t_2, bn2g, bn2b,
                w0_3, b0_3, w1_3, b1_3, w2t_3,
                out_ref):
    gall = g6[...]
    gj = jnp.broadcast_to(gall[:, None, :], (N, N, 6)).reshape(PB, 6)
    gi = jnp.broadcast_to(gall[None, :, :], (N, N, 6)).reshape(PB, 6)
    d = gi - gj
    r2 = jnp.dot(d * d, sel[...],
                 preferred_element_type=jnp.float32) + 1e-12
    r = jnp.sqrt(r2)
    ez = jnp.exp(jnp.concatenate([-4.0 * r2, 4.0 * r], axis=1))
    u = ez[:, 0:2]
    e1 = ez[:, 2:4]
    bB = u * e1 * C1
    bC = bB * e1 * C3
    basis = jnp.concatenate([u, bB, bC], axis=1)

    def conv(xt, w0, b0, w1, b1, w2t, d_out):
        h0 = _swish(jnp.dot(basis, w0[...],
                            preferred_element_type=jnp.float32) + b0[...])
        h1 = _swish(jnp.dot(h0, w1[...],
                            preferred_element_type=jnp.float32) + b1[...])
        yt_parts = []
        for bb in range(B):
            hb = h1[:, bb * H:(bb + 1) * H].reshape(N, N, H)
            hf = jnp.swapaxes(hb, 1, 2).reshape(N * H, N)
            m = jnp.dot(w2t[...], xt[:, bb * N:(bb + 1) * N],
                        preferred_element_type=jnp.float32)
            mt = jnp.swapaxes(m.reshape(d_out, H, N), 1, 2).reshape(d_out, N * H)
            yt_parts.append(jnp.dot(mt, hf,
                                    preferred_element_type=jnp.float32))
        return jnp.concatenate(yt_parts, axis=1)

    def bnorm(y, g, bta):
        mu = jnp.mean(y, axis=1, keepdims=True)
        dv = y - mu
        var = jnp.mean(dv * dv, axis=1, keepdims=True)
        return dv * jax.lax.rsqrt(var + 1e-5) * g[...] + bta[...]

    def gated(y):
        s = y[:MUL, :]
        gg = y[MUL:2 * MUL, :]
        ns = y[2 * MUL:, :]
        return jnp.concatenate([_swish(s), _sig(gg) * ns], axis=0)

    y = gated(bnorm(conv(x0t, w0_1, b0_1, w1_1, b1_1, w2t_1, D_MID), bn1g, bn1b))
    y = gated(bnorm(conv(y, w0_2, b0_2, w1_2, b1_2, w2t_2, D_MID), bn2g, bn2b))
    out_ref[...] = conv(y, w0_3, b0_3, w1_3, b1_3, w2t_3, D_OUT)


def _prep_w2(w2, d_out):
    return (w2.reshape(H, d_out, D_IN).transpose(1, 0, 2)
            .reshape(d_out * H, D_IN)) * SCALE


_EYE2 = np.eye(2, dtype=np.float32)


def _prep_w0(w0):
    return jnp.einsum('kh,bc->kbch', w0, _EYE2).reshape(2 * NB, 2 * H)


def _prep_w1(w1):
    return jnp.kron(jnp.asarray(_EYE2), w1)


def _prep_b(bvec):
    return jnp.tile(bvec, 2).reshape(1, 2 * H)


_SEL = np.kron(_EYE2, np.ones((3, 1), dtype=np.float32))


def kernel(input, geometry, r1_w0, r1_b0, r1_w1, r1_b1, r1_w2, bn1_g, bn1_b,
           r2_w0, r2_b0, r2_w1, r2_b1, r2_w2, bn2_g, bn2_b,
           r3_w0, r3_b0, r3_w1, r3_b1, r3_w2):
    g0 = geometry.astype(jnp.float32)
    g6 = jnp.concatenate([g0[0], g0[1]], axis=1)
    x0t = input.reshape(BN, D_IN).T

    args = (
        g6, x0t, jnp.asarray(_SEL),
        _prep_w0(r1_w0), _prep_b(r1_b0), _prep_w1(r1_w1), _prep_b(r1_b1),
        _prep_w2(r1_w2, D_MID), bn1_g.reshape(D_MID, 1), bn1_b.reshape(D_MID, 1),
        _prep_w0(r2_w0), _prep_b(r2_b0), _prep_w1(r2_w1), _prep_b(r2_b1),
        _prep_w2(r2_w2, D_MID), bn2_g.reshape(D_MID, 1), bn2_b.reshape(D_MID, 1),
        _prep_w0(r3_w0), _prep_b(r3_b0), _prep_w1(r3_w1), _prep_b(r3_b1),
        _prep_w2(r3_w2, D_OUT),
    )
    out = pl.pallas_call(
        _net_kernel,
        out_shape=jax.ShapeDtypeStruct((D_OUT, BN), jnp.float32),
    )(*args)
    return out.T.reshape(B, N, D_OUT)


# speedup vs baseline: 1.1472x; 1.1472x over previous
import jax
import jax.numpy as jnp
import numpy as np
from jax.experimental import pallas as pl

B, N, D_IN = 2, 96, 32
MUL = 16
NB, H = 3, 64
D_MID = 3 * MUL
D_OUT = 16
PB = N * N
BN = B * N
SCALE = 1.0 / np.sqrt(float(D_IN) * float(N))
C1 = float(np.exp(-1.0))
C3 = float(np.exp(-3.0))


def _sig(x):
    return 0.5 * jnp.tanh(0.5 * x) + 0.5


def _swish(x):
    return x * _sig(x)


def _bdiag2(w):
    z = jnp.zeros_like(w)
    return jnp.concatenate(
        [jnp.concatenate([w, z], axis=1), jnp.concatenate([z, w], axis=1)],
        axis=0)


def _net_kernel(g2, x0t,
                w0_1, b0_1, w1_1, b1_1, w2t_1, bn1g, bn1b,
                w0_2, b0_2, w1_2, b1_2, w2t_2, bn2g, bn2b,
                w0_3, b0_3, w1_3, b1_3, w2t_3,
                out_ref):
    gg = g2[...]
    g6 = jnp.concatenate([gg[:N, :], gg[N:, :]], axis=1)
    gj = jnp.broadcast_to(g6[:, None, :], (N, N, 6)).reshape(PB, 6)
    gi = jnp.broadcast_to(g6[None, :, :], (N, N, 6)).reshape(PB, 6)
    d = gi - gj
    ri = jax.lax.broadcasted_iota(jnp.int32, (6, 2), 0)
    ci = jax.lax.broadcasted_iota(jnp.int32, (6, 2), 1)
    sel = (ri // 3 == ci).astype(jnp.float32)
    r2 = jnp.dot(d * d, sel,
                 preferred_element_type=jnp.float32) + 1e-12
    r = jnp.sqrt(r2)
    ez = jnp.exp(jnp.concatenate([-4.0 * r2, 4.0 * r], axis=1))
    u = ez[:, 0:2]
    e1 = ez[:, 2:4]
    bB = u * e1 * C1
    bC = bB * e1 * C3
    basis = jnp.concatenate([u, bB, bC], axis=1)

    def conv(xt, w0, b0, w1, b1, w2t, d_out):
        z0 = jnp.zeros((NB, H), jnp.float32)
        wa = jnp.concatenate([w0[...], z0], axis=1)
        wb = jnp.concatenate([z0, w0[...]], axis=1)
        w0c = jnp.concatenate([wa[:, None, :], wb[:, None, :]],
                              axis=1).reshape(2 * NB, 2 * H)
        b0c = jnp.concatenate([b0[...], b0[...]], axis=1)
        w1c = _bdiag2(w1[...])
        b1c = jnp.concatenate([b1[...], b1[...]], axis=1)

        h0 = _swish(jnp.dot(basis, w0c,
                            preferred_element_type=jnp.float32) + b0c)
        h1 = _swish(jnp.dot(h0, w1c,
                            preferred_element_type=jnp.float32) + b1c)

        m = jnp.dot(w2t[...], xt,
                    preferred_element_type=jnp.float32)
        mt = jnp.swapaxes(m.reshape(d_out, H, BN), 1, 2) \
               .reshape(d_out, BN * H)
        yt_parts = []
        for bb in range(B):
            hb = h1[:, bb * H:(bb + 1) * H].reshape(N, N, H)
            hf = jnp.swapaxes(hb, 1, 2).reshape(N * H, N)
            mt_b = mt[:, bb * N * H:(bb + 1) * N * H]
            yt_parts.append(jnp.dot(mt_b, hf,
                                    preferred_element_type=jnp.float32))
        return jnp.concatenate(yt_parts, axis=1)

    def bnorm(y, g, bta):
        mu = jnp.mean(y, axis=1, keepdims=True)
        dv = y - mu
        var = jnp.mean(dv * dv, axis=1, keepdims=True)
        return dv * jax.lax.rsqrt(var + 1e-5) * g[...] + bta[...]

    def gated(y):
        s = y[:MUL, :]
        gg2 = y[MUL:2 * MUL, :]
        ns = y[2 * MUL:, :]
        return jnp.concatenate([_swish(s), _sig(gg2) * ns], axis=0)

    y = gated(bnorm(conv(x0t[...], w0_1, b0_1, w1_1, b1_1, w2t_1, D_MID), bn1g, bn1b))
    y = gated(bnorm(conv(y, w0_2, b0_2, w1_2, b1_2, w2t_2, D_MID), bn2g, bn2b))
    out_ref[...] = conv(y, w0_3, b0_3, w1_3, b1_3, w2t_3, D_OUT)


def _prep_w2(w2, d_out):
    return (w2.reshape(H, d_out, D_IN).transpose(1, 0, 2)
            .reshape(d_out * H, D_IN)) * SCALE


def kernel(input, geometry, r1_w0, r1_b0, r1_w1, r1_b1, r1_w2, bn1_g, bn1_b,
           r2_w0, r2_b0, r2_w1, r2_b1, r2_w2, bn2_g, bn2_b,
           r3_w0, r3_b0, r3_w1, r3_b1, r3_w2):
    g2 = geometry.astype(jnp.float32).reshape(BN, 3)
    x0t = input.reshape(BN, D_IN).T

    args = (
        g2, x0t,
        r1_w0, r1_b0.reshape(1, H), r1_w1, r1_b1.reshape(1, H),
        _prep_w2(r1_w2, D_MID), bn1_g.reshape(D_MID, 1), bn1_b.reshape(D_MID, 1),
        r2_w0, r2_b0.reshape(1, H), r2_w1, r2_b1.reshape(1, H),
        _prep_w2(r2_w2, D_MID), bn2_g.reshape(D_MID, 1), bn2_b.reshape(D_MID, 1),
        r3_w0, r3_b0.reshape(1, H), r3_w1, r3_b1.reshape(1, H),
        _prep_w2(r3_w2, D_OUT),
    )
    out = pl.pallas_call(
        _net_kernel,
        out_shape=jax.ShapeDtypeStruct((D_OUT, BN), jnp.float32),
    )(*args)
    return out.T.reshape(B, N, D_OUT)


# speedup vs baseline: 1.4211x; 1.2387x over previous
import jax
import jax.numpy as jnp
import numpy as np
from jax.experimental import pallas as pl

B, N, D_IN = 2, 96, 32
MUL = 16
NB, H = 3, 64
D_MID = 3 * MUL
D_OUT = 16
PB = N * N
BN = B * N
SCALE = 1.0 / np.sqrt(float(D_IN) * float(N))

_WA = np.zeros((5, 7), np.float32)
for _k in range(NB):
    for _b in range(2):
        _c = 2 * _k + _b
        _WA[_b, _c] = -4.0
        _WA[2 + _b, _c] = 4.0 * _k
        _WA[4, _c] = -float(_k * _k)


def _sig(x):
    return 1.0 / (1.0 + jnp.exp(-x))


def _swish(x):
    return x * _sig(x)


def _bdiag2(w):
    z = jnp.zeros_like(w)
    return jnp.concatenate(
        [jnp.concatenate([w, z], axis=1), jnp.concatenate([z, w], axis=1)],
        axis=0)


def _net_kernel(g2, x0t, wa_m,
                w0_1, b0_1, w1_1, b1_1, w2t_1, bn1g, bn1b,
                w0_2, b0_2, w1_2, b1_2, w2t_2, bn2g, bn2b,
                w0_3, b0_3, w1_3, b1_3, w2t_3,
                out_ref):
    gg = g2[...]
    g6 = jnp.concatenate([gg[:N, :], gg[N:, :]], axis=1)
    gj = jnp.broadcast_to(g6[:, None, :], (N, N, 6)).reshape(PB, 6)
    gi = jnp.broadcast_to(g6[None, :, :], (N, N, 6)).reshape(PB, 6)
    d = gi - gj
    ri = jax.lax.broadcasted_iota(jnp.int32, (6, 2), 0)
    ci = jax.lax.broadcasted_iota(jnp.int32, (6, 2), 1)
    sel = (ri // 3 == ci).astype(jnp.float32)
    r2 = jnp.dot(d * d, sel,
                 preferred_element_type=jnp.float32) + 1e-12
    r = jnp.sqrt(r2)
    q = jnp.concatenate([r2, r, jnp.ones((PB, 1), jnp.float32)], axis=1)
    basis_aug = jnp.exp(jnp.dot(q, wa_m[...],
                                preferred_element_type=jnp.float32))

    def conv(xt, w0, b0, w1, b1, w2t, d_out):
        z0 = jnp.zeros((NB, H), jnp.float32)
        wa = jnp.concatenate([w0[...], z0], axis=1)
        wb = jnp.concatenate([z0, w0[...]], axis=1)
        b0c = jnp.concatenate([b0[...], b0[...]], axis=1)
        w0c = jnp.concatenate(
            [jnp.concatenate([wa[:, None, :], wb[:, None, :]],
                             axis=1).reshape(2 * NB, 2 * H), b0c], axis=0)
        w1c = _bdiag2(w1[...])
        b1c = jnp.concatenate([b1[...], b1[...]], axis=1)

        h0 = _swish(jnp.dot(basis_aug, w0c,
                            preferred_element_type=jnp.float32))
        h1 = _swish(jnp.dot(h0, w1c,
                            preferred_element_type=jnp.float32) + b1c
                    ).astype(jnp.bfloat16)

        m = jnp.dot(w2t[...], xt,
                    preferred_element_type=jnp.float32
                    ).astype(jnp.bfloat16)
        mt = jnp.swapaxes(m.reshape(d_out, H, BN), 1, 2) \
               .reshape(d_out, BN * H)
        yt_parts = []
        for bb in range(B):
            hb = h1[:, bb * H:(bb + 1) * H].reshape(N, N, H)
            hf = jnp.swapaxes(hb, 1, 2).reshape(N * H, N)
            mt_b = mt[:, bb * N * H:(bb + 1) * N * H]
            yt_parts.append(jnp.dot(mt_b, hf,
                                    preferred_element_type=jnp.float32))
        return jnp.concatenate(yt_parts, axis=1)

    def bnorm(y, g, bta):
        mu = jnp.mean(y, axis=1, keepdims=True)
        dv = y - mu
        var = jnp.mean(dv * dv, axis=1, keepdims=True)
        return dv * jax.lax.rsqrt(var + 1e-5) * g[...] + bta[...]

    def gated(y):
        s = y[:MUL, :]
        gg2 = y[MUL:2 * MUL, :]
        ns = y[2 * MUL:, :]
        return jnp.concatenate([_swish(s), _sig(gg2) * ns], axis=0)

    y = gated(bnorm(conv(x0t[...], w0_1, b0_1, w1_1, b1_1, w2t_1, D_MID), bn1g, bn1b))
    y = gated(bnorm(conv(y, w0_2, b0_2, w1_2, b1_2, w2t_2, D_MID), bn2g, bn2b))
    out_ref[...] = conv(y, w0_3, b0_3, w1_3, b1_3, w2t_3, D_OUT)


def _prep_w2(w2, d_out):
    return (w2.reshape(H, d_out, D_IN).transpose(1, 0, 2)
            .reshape(d_out * H, D_IN)) * SCALE


def kernel(input, geometry, r1_w0, r1_b0, r1_w1, r1_b1, r1_w2, bn1_g, bn1_b,
           r2_w0, r2_b0, r2_w1, r2_b1, r2_w2, bn2_g, bn2_b,
           r3_w0, r3_b0, r3_w1, r3_b1, r3_w2):
    g2 = geometry.astype(jnp.float32).reshape(BN, 3)
    x0t = input.reshape(BN, D_IN).T

    args = (
        g2, x0t, jnp.asarray(_WA),
        r1_w0, r1_b0.reshape(1, H), r1_w1, r1_b1.reshape(1, H),
        _prep_w2(r1_w2, D_MID), bn1_g.reshape(D_MID, 1), bn1_b.reshape(D_MID, 1),
        r2_w0, r2_b0.reshape(1, H), r2_w1, r2_b1.reshape(1, H),
        _prep_w2(r2_w2, D_MID), bn2_g.reshape(D_MID, 1), bn2_b.reshape(D_MID, 1),
        r3_w0, r3_b0.reshape(1, H), r3_w1, r3_b1.reshape(1, H),
        _prep_w2(r3_w2, D_OUT),
    )
    out = pl.pallas_call(
        _net_kernel,
        out_shape=jax.ShapeDtypeStruct((D_OUT, BN), jnp.float32),
    )(*args)
    return out.T.reshape(B, N, D_OUT)


# speedup vs baseline: 1.4406x; 1.0137x over previous
import jax
import jax.numpy as jnp
import numpy as np
from jax.experimental import pallas as pl

B, N, D_IN = 2, 96, 32
MUL = 16
NB, H = 3, 64
D_MID = 3 * MUL
D_OUT = 16
PB = N * N
BN = B * N
SCALE = 1.0 / np.sqrt(float(D_IN) * float(N))

_WA = np.zeros((5, 7), np.float32)
for _k in range(NB):
    for _b in range(2):
        _c = 2 * _k + _b
        _WA[_b, _c] = -4.0
        _WA[2 + _b, _c] = 4.0 * _k
        _WA[4, _c] = -float(_k * _k)


_NLOG2E = -1.4426950408889634


def _sig(x):
    return 1.0 / (1.0 + jnp.exp2(x * _NLOG2E))


def _swish(x):
    return x * _sig(x)


def _bdiag2(w):
    z = jnp.zeros_like(w)
    return jnp.concatenate(
        [jnp.concatenate([w, z], axis=1), jnp.concatenate([z, w], axis=1)],
        axis=0)


def _net_kernel(g2, x0t, wa_m,
                w0_1, b0_1, w1_1, b1_1, w2t_1, bn1g, bn1b,
                w0_2, b0_2, w1_2, b1_2, w2t_2, bn2g, bn2b,
                w0_3, b0_3, w1_3, b1_3, w2t_3,
                out_ref):
    gg = g2[...]
    g6 = jnp.concatenate([gg[:N, :], gg[N:, :]], axis=1)
    gj = jnp.broadcast_to(g6[:, None, :], (N, N, 6)).reshape(PB, 6)
    gi = jnp.broadcast_to(g6[None, :, :], (N, N, 6)).reshape(PB, 6)
    d = gi - gj
    ri = jax.lax.broadcasted_iota(jnp.int32, (6, 2), 0)
    ci = jax.lax.broadcasted_iota(jnp.int32, (6, 2), 1)
    sel = (ri // 3 == ci).astype(jnp.float32)
    r2 = jnp.dot(d * d, sel,
                 preferred_element_type=jnp.float32) + 1e-12
    r = r2 * jax.lax.rsqrt(r2)
    q = jnp.concatenate([r2, r, jnp.ones((PB, 1), jnp.float32)], axis=1)
    basis_aug = jnp.exp(jnp.dot(q, wa_m[...],
                                preferred_element_type=jnp.float32))

    def conv(xt, w0, b0, w1, b1, w2t, d_out):
        z0 = jnp.zeros((NB, H), jnp.float32)
        wa = jnp.concatenate([w0[...], z0], axis=1)
        wb = jnp.concatenate([z0, w0[...]], axis=1)
        b0c = jnp.concatenate([b0[...], b0[...]], axis=1)
        w0c = jnp.concatenate(
            [jnp.concatenate([wa[:, None, :], wb[:, None, :]],
                             axis=1).reshape(2 * NB, 2 * H), b0c], axis=0)
        w1c = _bdiag2(w1[...])
        b1c = jnp.concatenate([b1[...], b1[...]], axis=1)

        h0 = _swish(jnp.dot(basis_aug, w0c,
                            preferred_element_type=jnp.float32))
        h1 = _swish(jnp.dot(h0, w1c,
                            preferred_element_type=jnp.float32) + b1c
                    ).astype(jnp.bfloat16)

        m = jnp.dot(w2t[...], xt,
                    preferred_element_type=jnp.float32
                    ).astype(jnp.bfloat16)
        mt = jnp.swapaxes(m.reshape(d_out, H, BN), 1, 2) \
               .reshape(d_out, BN * H)
        yt_parts = []
        for bb in range(B):
            hb = h1[:, bb * H:(bb + 1) * H].reshape(N, N, H)
            hf = jnp.swapaxes(hb, 1, 2).reshape(N * H, N)
            mt_b = mt[:, bb * N * H:(bb + 1) * N * H]
            yt_parts.append(jnp.dot(mt_b, hf,
                                    preferred_element_type=jnp.float32))
        return jnp.concatenate(yt_parts, axis=1)

    def bnorm(y, g, bta):
        mu = jnp.mean(y, axis=1, keepdims=True)
        dv = y - mu
        var = jnp.mean(dv * dv, axis=1, keepdims=True)
        return dv * jax.lax.rsqrt(var + 1e-5) * g[...] + bta[...]

    def gated(y):
        s = y[:MUL, :]
        gg2 = y[MUL:2 * MUL, :]
        ns = y[2 * MUL:, :]
        return jnp.concatenate([_swish(s), _sig(gg2) * ns], axis=0)

    y = gated(bnorm(conv(x0t[...], w0_1, b0_1, w1_1, b1_1, w2t_1, D_MID), bn1g, bn1b))
    y = gated(bnorm(conv(y, w0_2, b0_2, w1_2, b1_2, w2t_2, D_MID), bn2g, bn2b))
    out_ref[...] = conv(y, w0_3, b0_3, w1_3, b1_3, w2t_3, D_OUT)


def _prep_w2(w2, d_out):
    return (w2.reshape(H, d_out, D_IN).transpose(1, 0, 2)
            .reshape(d_out * H, D_IN)) * SCALE


def kernel(input, geometry, r1_w0, r1_b0, r1_w1, r1_b1, r1_w2, bn1_g, bn1_b,
           r2_w0, r2_b0, r2_w1, r2_b1, r2_w2, bn2_g, bn2_b,
           r3_w0, r3_b0, r3_w1, r3_b1, r3_w2):
    g2 = geometry.astype(jnp.float32).reshape(BN, 3)
    x0t = input.reshape(BN, D_IN).T

    args = (
        g2, x0t, jnp.asarray(_WA),
        r1_w0, r1_b0.reshape(1, H), r1_w1, r1_b1.reshape(1, H),
        _prep_w2(r1_w2, D_MID), bn1_g.reshape(D_MID, 1), bn1_b.reshape(D_MID, 1),
        r2_w0, r2_b0.reshape(1, H), r2_w1, r2_b1.reshape(1, H),
        _prep_w2(r2_w2, D_MID), bn2_g.reshape(D_MID, 1), bn2_b.reshape(D_MID, 1),
        r3_w0, r3_b0.reshape(1, H), r3_w1, r3_b1.reshape(1, H),
        _prep_w2(r3_w2, D_OUT),
    )
    out = pl.pallas_call(
        _net_kernel,
        out_shape=jax.ShapeDtypeStruct((D_OUT, BN), jnp.float32),
    )(*args)
    return out.T.reshape(B, N, D_OUT)
